# Initial kernel scaffold; baseline (speedup 1.0000x reference)
import jax
import jax.numpy as jnp
from jax.experimental import pallas as pl


def kernel(x, edge_index, mask):
    raise NotImplementedError("write your pallas kernel here")


# speedup vs baseline: 1.0000x; 1.0000x over previous
import functools
import math

import jax
import jax.numpy as jnp
from jax.experimental import pallas as pl

_N = 10000
_D = 128
_ALPHA = 0.9
_BETA = 0.5
_NUM_ITERS = 20
_BFS_STEPS = 10
_LN_ALPHA = math.log(_ALPHA)


def _corr_body(out_ref, f_ref, o_ref):
    out = out_ref[...]
    f = f_ref[...]
    n, d = out.shape
    mean = jnp.mean(out, axis=0, keepdims=True)
    xm = out - mean
    cov = jnp.dot(xm.T, xm, preferred_element_type=jnp.float32) / (n - 1)
    eye = (jax.lax.broadcasted_iota(jnp.int32, (d, d), 0)
           == jax.lax.broadcasted_iota(jnp.int32, (d, d), 1))
    var = jnp.sum(jnp.where(eye, cov, 0.0), axis=1)
    std = jnp.sqrt(var)
    denom = std[:, None] * std[None, :]
    cor = jnp.where(denom > 0, cov / denom, 0.0)
    cor = jnp.where(eye, 0.0, cor)
    af = jnp.exp(f * _LN_ALPHA)
    a_1 = af * xm
    a_2 = jnp.dot(a_1, cor, preferred_element_type=jnp.float32)
    o_ref[...] = out + _BETA * (1.0 - af) * a_2


def _corr_correction(out, f):
    return pl.pallas_call(
        _corr_body,
        out_shape=jax.ShapeDtypeStruct(out.shape, jnp.float32),
    )(out, f.reshape(-1, 1))


def kernel(x, edge_index, mask):
    n, d = x.shape
    row, col = edge_index[0], edge_index[1]
    big = n + 1
    dist = jnp.where(mask[:, 0], 0, big).astype(jnp.int32)
    for _ in range(_BFS_STEPS):
        dist = dist.at[row].min(dist[col] + 1)
    f = jnp.where(dist >= big, 0, dist).astype(jnp.float32)

    w = _ALPHA ** (f[col] - f[row] + 1.0)
    deg = jax.ops.segment_sum(w, row, num_segments=n)
    deg_inv = jnp.where(deg > 0, 1.0 / deg, 0.0)
    a_vals = w * deg_inv[row]

    out = jnp.where(mask, x, 0.0)
    for _ in range(_NUM_ITERS):
        out = jax.ops.segment_sum(a_vals[:, None] * out[col], row,
                                  num_segments=n)
        out = jnp.where(mask, x, out)

    return _corr_correction(out, f)


# speedup vs baseline: 8.3625x; 8.3624x over previous
import functools
import math

import jax
import jax.numpy as jnp
from jax import lax
from jax.experimental import pallas as pl
from jax.experimental.pallas import tpu as pltpu
from jax.experimental.pallas import tpu_sc as plsc

_N = 10000
_D = 128
_E = 320000
_ALPHA = 0.9
_BETA = 0.5
_NUM_ITERS = 20
_BFS_STEPS = 10
_LN_ALPHA = math.log(_ALPHA)
_BIG = _N + 1

_H = _D // 2
_NT = 16
_EPT = _E // _NT
_EB = 400
_NB = _EPT // _EB
_NP = 10240
_RPT = _NP // _NT
_RC = 128
_NCH = _RPT // _RC


def _iota16():
    return lax.iota(jnp.int32, 16)


def _sc_body(xm2_h, nm2_h, row_h, col_h, seed_h, out2_h, f_h, av_h,
             dist, fv, ncopy, gath, col_b, row_b, a_b, val_b, w_b,
             tbuf, acc_c, xm_c, nm_c, zb, zbf,
             accum, nacc, deg_s, sem):
    c = lax.axis_index("c")
    s = lax.axis_index("s")
    base_e = s * _EPT

    def _fill_zb(i, _):
        zb[pl.ds(i * 16, 16)] = jnp.zeros((16,), jnp.int32)
        zbf[pl.ds(i * 16, 16)] = jnp.zeros((16,), jnp.float32)
        return 0
    lax.fori_loop(0, 640 // 16, _fill_zb, 0)

    pltpu.sync_copy(seed_h, dist.at[pl.ds(0, _N)])

    def _init_dist(i, _):
        sv = dist[pl.ds(i * 16, 16)]
        dist[pl.ds(i * 16, 16)] = (1 - sv) * _BIG
        return 0
    lax.fori_loop(0, _N // 16, _init_dist, 0)
    def _pad_dist(i, _):
        dist[pl.ds(_N + i * 16, 16)] = jnp.full((16,), _BIG, jnp.int32)
        return 0
    lax.fori_loop(0, (_NP - _N) // 16, _pad_dist, 0)

    pltpu.sync_copy(zb, nacc.at[pl.ds(s * 640, 640)])
    plsc.subcore_barrier()

    def _bfs_step(step, _):
        def _batch(k, _):
            off = base_e + k * _EB
            pltpu.sync_copy(col_h.at[pl.ds(off, _EB)], col_b)
            pltpu.sync_copy(row_h.at[pl.ds(off, _EB)], row_b)
            def _grp(g, _):
                cv = col_b[pl.ds(g * 16, 16)]
                dcol = plsc.load_gather(dist, [cv])
                val_b[pl.ds(g * 16, 16)] = jnp.where(
                    dcol < _BIG, 1, 0).astype(jnp.int32)
                return 0
            lax.fori_loop(0, _EB // 16, _grp, 0)
            pltpu.sync_copy(val_b, nacc.at[row_b], add=True)
            return 0
        lax.fori_loop(0, _NB, _batch, 0)
        plsc.subcore_barrier()
        pltpu.sync_copy(nacc, ncopy)
        plsc.subcore_barrier()
        pltpu.sync_copy(zb, nacc.at[pl.ds(s * 640, 640)])
        def _upd(i, _):
            av = ncopy[pl.ds(i * 16, 16)]
            dv = dist[pl.ds(i * 16, 16)]
            newly = (av > 0) & (dv >= _BIG)
            dist[pl.ds(i * 16, 16)] = jnp.where(newly, step + 1, dv)
            return 0
        lax.fori_loop(0, _NP // 16, _upd, 0)
        plsc.subcore_barrier()
        return 0
    lax.fori_loop(0, _BFS_STEPS, _bfs_step, 0)

    def _mk_f(i, _):
        dv = dist[pl.ds(i * 16, 16)]
        fv[pl.ds(i * 16, 16)] = jnp.where(
            dv >= _BIG, 0, dv).astype(jnp.float32)
        return 0
    lax.fori_loop(0, _NP // 16, _mk_f, 0)

    @pl.when((c == 0) & (s < 10))
    def _():
        pltpu.sync_copy(fv.at[pl.ds(s * 1000, 1000)],
                        f_h.at[pl.ds(s * 1000, 1000)])

    pltpu.sync_copy(zbf, deg_s.at[pl.ds(s * 640, 640)])
    plsc.subcore_barrier()

    def _wbatch(k, _):
        off = base_e + k * _EB
        pltpu.sync_copy(col_h.at[pl.ds(off, _EB)], col_b)
        pltpu.sync_copy(row_h.at[pl.ds(off, _EB)], row_b)
        def _grp(g, _):
            cv = col_b[pl.ds(g * 16, 16)]
            rv = row_b[pl.ds(g * 16, 16)]
            fc = plsc.load_gather(fv, [cv])
            fr = plsc.load_gather(fv, [rv])
            w_b[pl.ds(g * 16, 16)] = jnp.exp(
                _LN_ALPHA * (fc - fr + 1.0))
            return 0
        lax.fori_loop(0, _EB // 16, _grp, 0)
        pltpu.sync_copy(w_b, deg_s.at[row_b], add=True)
        pltpu.sync_copy(w_b, av_h.at[pl.ds(c * _E + off, _EB)])
        return 0
    lax.fori_loop(0, _NB, _wbatch, 0)
    plsc.subcore_barrier()

    pltpu.sync_copy(deg_s.at[pl.ds(s * 640, 640)], tbuf)
    def _dinv(i, _):
        dv = tbuf[pl.ds(i * 16, 16)]
        tbuf[pl.ds(i * 16, 16)] = jnp.where(dv > 0.0, 1.0 / dv, 0.0)
        return 0
    lax.fori_loop(0, 640 // 16, _dinv, 0)
    pltpu.sync_copy(tbuf, deg_s.at[pl.ds(s * 640, 640)])
    plsc.subcore_barrier()

    pltpu.sync_copy(deg_s, fv)
    def _abatch(k, _):
        off = base_e + k * _EB
        pltpu.sync_copy(row_h.at[pl.ds(off, _EB)], row_b)
        pltpu.sync_copy(av_h.at[pl.ds(c * _E + off, _EB)], w_b)
        def _grp(g, _):
            rv = row_b[pl.ds(g * 16, 16)]
            di = plsc.load_gather(fv, [rv])
            w_b[pl.ds(g * 16, 16)] = w_b[pl.ds(g * 16, 16)] * di
            return 0
        lax.fori_loop(0, _EB // 16, _grp, 0)
        pltpu.sync_copy(w_b, av_h.at[pl.ds(c * _E + off, _EB)])
        return 0
    lax.fori_loop(0, _NB, _abatch, 0)

    def _init_out(j, _):
        r0 = c * _NP + s * _RPT + j * _RC
        pltpu.sync_copy(xm2_h.at[pl.ds(r0, _RC)], xm_c)
        pltpu.sync_copy(xm_c, out2_h.at[pl.ds(r0, _RC)])
        return 0
    lax.fori_loop(0, _NCH, _init_out, 0)
    plsc.subcore_barrier()

    def _iter(t, _):
        def _zrow(r, _):
            idx_r = jnp.full((16,), r, jnp.int32)
            for q in range(_H // 16):
                idx_c = _iota16() + q * 16
                plsc.store_scatter(acc_c, [idx_r, idx_c],
                                   jnp.zeros((16,), jnp.float32))
            return 0
        lax.fori_loop(0, _RC, _zrow, 0)
        def _zch(j, _):
            pltpu.sync_copy(acc_c, accum.at[pl.ds(s * _RPT + j * _RC, _RC)])
            return 0
        lax.fori_loop(0, _NCH, _zch, 0)
        plsc.subcore_barrier()

        def _batch(k, _):
            off = base_e + k * _EB
            pltpu.sync_copy(col_h.at[pl.ds(off, _EB)], col_b)
            pltpu.sync_copy(row_h.at[pl.ds(off, _EB)], row_b)
            pltpu.sync_copy(av_h.at[pl.ds(c * _E + off, _EB)], a_b)
            def _adj(g, _):
                col_b[pl.ds(g * 16, 16)] = (
                    col_b[pl.ds(g * 16, 16)] + c * _NP)
                return 0
            lax.fori_loop(0, _EB // 16, _adj, 0)
            pltpu.async_copy(out2_h.at[col_b], gath, sem).wait()
            def _scale(g, _):
                av = a_b[pl.ds(g * 16, 16)]
                idx_r = g * 16 + _iota16()
                for q in range(_H // 16):
                    idx_c = _iota16() + q * 16
                    v = plsc.load_gather(gath, [idx_r, idx_c])
                    plsc.store_scatter(gath, [idx_r, idx_c], v * av)
                return 0
            lax.fori_loop(0, _EB // 16, _scale, 0)
            pltpu.sync_copy(gath, accum.at[row_b], add=True)
            return 0
        lax.fori_loop(0, _NB, _batch, 0)
        plsc.subcore_barrier()

        def _cmb(j, _):
            r0 = s * _RPT + j * _RC
            g0 = c * _NP + r0
            pltpu.sync_copy(accum.at[pl.ds(r0, _RC)], acc_c)
            pltpu.sync_copy(xm2_h.at[pl.ds(g0, _RC)], xm_c)
            pltpu.sync_copy(nm2_h.at[pl.ds(g0, _RC)], nm_c)
            def _crow(r, _):
                idx_r = jnp.full((16,), r, jnp.int32)
                for q in range(_H // 16):
                    idx_c = _iota16() + q * 16
                    a = plsc.load_gather(acc_c, [idx_r, idx_c])
                    x = plsc.load_gather(xm_c, [idx_r, idx_c])
                    m = plsc.load_gather(nm_c, [idx_r, idx_c])
                    plsc.store_scatter(acc_c, [idx_r, idx_c], x + m * a)
                return 0
            lax.fori_loop(0, _RC, _crow, 0)
            pltpu.sync_copy(acc_c, out2_h.at[pl.ds(g0, _RC)])
            return 0
        lax.fori_loop(0, _NCH, _cmb, 0)
        plsc.subcore_barrier()
        return 0
    lax.fori_loop(0, _NUM_ITERS, _iter, 0)


def _sc_propagate(xm2, nm2, row, col, seed):
    mesh = plsc.VectorSubcoreMesh(core_axis_name="c", subcore_axis_name="s")
    fn = functools.partial(
        pl.kernel, _sc_body, mesh=mesh,
        compiler_params=pltpu.CompilerParams(needs_layout_passes=False,
                                             use_tc_tiling_on_sc=False),
        out_type=[
            jax.ShapeDtypeStruct((2 * _NP, _H), jnp.float32),
            jax.ShapeDtypeStruct((_N,), jnp.float32),
            jax.ShapeDtypeStruct((2 * _E,), jnp.float32),
        ],
        scratch_types=[
            pltpu.VMEM((_NP,), jnp.int32),
            pltpu.VMEM((_NP,), jnp.float32),
            pltpu.VMEM((_NP,), jnp.int32),
            pltpu.VMEM((_EB, _H), jnp.float32),
            pltpu.VMEM((_EB,), jnp.int32),
            pltpu.VMEM((_EB,), jnp.int32),
            pltpu.VMEM((_EB,), jnp.float32),
            pltpu.VMEM((_EB,), jnp.int32),
            pltpu.VMEM((_EB,), jnp.float32),
            pltpu.VMEM((640,), jnp.float32),
            pltpu.VMEM((_RC, _H), jnp.float32),
            pltpu.VMEM((_RC, _H), jnp.float32),
            pltpu.VMEM((_RC, _H), jnp.float32),
            pltpu.VMEM((640,), jnp.int32),
            pltpu.VMEM((640,), jnp.float32),
            pltpu.VMEM_SHARED((_NP, _H), jnp.float32),
            pltpu.VMEM_SHARED((_NP,), jnp.int32),
            pltpu.VMEM_SHARED((_NP,), jnp.float32),
            pltpu.SemaphoreType.DMA,
        ],
    )()
    out2, f, _ = fn(xm2, nm2, row, col, seed)
    return out2, f


def _corr_body(out_ref, f_ref, o_ref):
    out = out_ref[...]
    f = f_ref[...]
    n, d = out.shape
    mean = jnp.mean(out, axis=0, keepdims=True)
    xm = out - mean
    cov = jnp.dot(xm.T, xm, preferred_element_type=jnp.float32) / (n - 1)
    eye = (lax.broadcasted_iota(jnp.int32, (d, d), 0)
           == lax.broadcasted_iota(jnp.int32, (d, d), 1))
    var = jnp.sum(jnp.where(eye, cov, 0.0), axis=1)
    std = jnp.sqrt(var)
    denom = std[:, None] * std[None, :]
    cor = jnp.where(denom > 0, cov / denom, 0.0)
    cor = jnp.where(eye, 0.0, cor)
    af = jnp.exp(f * _LN_ALPHA)
    a_1 = af * xm
    a_2 = jnp.dot(a_1, cor, preferred_element_type=jnp.float32)
    o_ref[...] = out + _BETA * (1.0 - af) * a_2


def _corr_correction(out, f):
    return pl.pallas_call(
        _corr_body,
        out_shape=jax.ShapeDtypeStruct(out.shape, jnp.float32),
    )(out, f.reshape(-1, 1))


def kernel(x, edge_index, mask):
    row = edge_index[0]
    col = edge_index[1]
    xm = jnp.where(mask, x, 0.0)
    nm = (~mask).astype(jnp.float32)
    pad = jnp.zeros((_NP - _N, _H), jnp.float32)
    xm2 = jnp.concatenate([xm[:, :_H], pad, xm[:, _H:], pad], axis=0)
    nm2 = jnp.concatenate([nm[:, :_H], pad, nm[:, _H:], pad], axis=0)
    seed = mask[:, 0].astype(jnp.int32)
    out2, f = _sc_propagate(xm2, nm2, row, col, seed)
    out = jnp.concatenate([out2[:_N], out2[_NP:_NP + _N]], axis=1)
    return _corr_correction(out, f)


# speedup vs baseline: 8.6927x; 1.0395x over previous
import functools
import math

import jax
import jax.numpy as jnp
from jax import lax
from jax.experimental import pallas as pl
from jax.experimental.pallas import tpu as pltpu
from jax.experimental.pallas import tpu_sc as plsc

_N = 10000
_D = 128
_E = 320000
_ALPHA = 0.9
_BETA = 0.5
_NUM_ITERS = 20
_BFS_STEPS = 10
_LN_ALPHA = math.log(_ALPHA)
_BIG = _N + 1

_H = _D // 2
_NT = 16
_EPT = _E // _NT
_EB = 200
_NB = _EPT // _EB
_NP = 10240
_RPT = _NP // _NT
_RC = 64
_NCH = _RPT // _RC


def _iota16():
    return lax.iota(jnp.int32, 16)


def _sc_body(xm2_h, nm2_h, row_h, col_h, seed_h, out2_h, f_h, av_h,
             dist, fv, ncopy,
             gath0, gath1, col_b0, col_b1, row_b0, row_b1, a_b0, a_b1,
             val_b, w_b, tbuf, acc_c, xm_c, nm_c, zf, zb, zbf,
             accum, nacc, deg_s,
             sem, sem_g0, sem_g1, sem_i0, sem_i1, sem_s0, sem_s1):
    c = lax.axis_index("c")
    s = lax.axis_index("s")
    base_e = s * _EPT
    col_b, row_b, a_b = col_b0, row_b0, a_b0

    def _fill_zb(i, _):
        zb[pl.ds(i * 16, 16)] = jnp.zeros((16,), jnp.int32)
        zbf[pl.ds(i * 16, 16)] = jnp.zeros((16,), jnp.float32)
        return 0
    lax.fori_loop(0, 640 // 16, _fill_zb, 0)
    def _fill_zf(r, _):
        idx_r = jnp.full((16,), r, jnp.int32)
        for q in range(_H // 16):
            idx_c = _iota16() + q * 16
            plsc.store_scatter(zf, [idx_r, idx_c],
                               jnp.zeros((16,), jnp.float32))
        return 0
    lax.fori_loop(0, _RC, _fill_zf, 0)

    pltpu.sync_copy(seed_h, dist.at[pl.ds(0, _N)])

    def _init_dist(i, _):
        sv = dist[pl.ds(i * 16, 16)]
        dist[pl.ds(i * 16, 16)] = (1 - sv) * _BIG
        return 0
    lax.fori_loop(0, _N // 16, _init_dist, 0)
    def _pad_dist(i, _):
        dist[pl.ds(_N + i * 16, 16)] = jnp.full((16,), _BIG, jnp.int32)
        return 0
    lax.fori_loop(0, (_NP - _N) // 16, _pad_dist, 0)

    pltpu.sync_copy(zb, nacc.at[pl.ds(s * 640, 640)])
    plsc.subcore_barrier()

    def _bfs_step(step, _):
        def _batch(k, _):
            off = base_e + k * _EB
            pltpu.sync_copy(col_h.at[pl.ds(off, _EB)], col_b)
            pltpu.sync_copy(row_h.at[pl.ds(off, _EB)], row_b)
            def _grp(g, _):
                cv = col_b[pl.ds(g * 16, 16)]
                dcol = plsc.load_gather(dist, [cv])
                val_b[pl.ds(g * 16, 16)] = jnp.where(
                    dcol < _BIG, 1, 0).astype(jnp.int32)
                return 0
            lax.fori_loop(0, _EB // 16, _grp, 0)
            pltpu.sync_copy(val_b, nacc.at[row_b], add=True)
            return 0
        lax.fori_loop(0, _NB, _batch, 0)
        plsc.subcore_barrier()
        pltpu.sync_copy(nacc, ncopy)
        plsc.subcore_barrier()
        pltpu.sync_copy(zb, nacc.at[pl.ds(s * 640, 640)])
        def _upd(i, _):
            av = ncopy[pl.ds(i * 16, 16)]
            dv = dist[pl.ds(i * 16, 16)]
            newly = (av > 0) & (dv >= _BIG)
            dist[pl.ds(i * 16, 16)] = jnp.where(newly, step + 1, dv)
            return 0
        lax.fori_loop(0, _NP // 16, _upd, 0)
        plsc.subcore_barrier()
        return 0
    lax.fori_loop(0, _BFS_STEPS, _bfs_step, 0)

    def _mk_f(i, _):
        dv = dist[pl.ds(i * 16, 16)]
        fv[pl.ds(i * 16, 16)] = jnp.where(
            dv >= _BIG, 0, dv).astype(jnp.float32)
        return 0
    lax.fori_loop(0, _NP // 16, _mk_f, 0)

    @pl.when((c == 0) & (s < 10))
    def _():
        pltpu.sync_copy(fv.at[pl.ds(s * 1000, 1000)],
                        f_h.at[pl.ds(s * 1000, 1000)])

    pltpu.sync_copy(zbf, deg_s.at[pl.ds(s * 640, 640)])
    plsc.subcore_barrier()

    def _wbatch(k, _):
        off = base_e + k * _EB
        pltpu.sync_copy(col_h.at[pl.ds(off, _EB)], col_b)
        pltpu.sync_copy(row_h.at[pl.ds(off, _EB)], row_b)
        def _grp(g, _):
            cv = col_b[pl.ds(g * 16, 16)]
            rv = row_b[pl.ds(g * 16, 16)]
            fc = plsc.load_gather(fv, [cv])
            fr = plsc.load_gather(fv, [rv])
            w_b[pl.ds(g * 16, 16)] = jnp.exp(
                _LN_ALPHA * (fc - fr + 1.0))
            return 0
        lax.fori_loop(0, _EB // 16, _grp, 0)
        pltpu.sync_copy(w_b, deg_s.at[row_b], add=True)
        pltpu.sync_copy(w_b, av_h.at[pl.ds(c * _E + off, _EB)])
        return 0
    lax.fori_loop(0, _NB, _wbatch, 0)
    plsc.subcore_barrier()

    pltpu.sync_copy(deg_s.at[pl.ds(s * 640, 640)], tbuf)
    def _dinv(i, _):
        dv = tbuf[pl.ds(i * 16, 16)]
        tbuf[pl.ds(i * 16, 16)] = jnp.where(dv > 0.0, 1.0 / dv, 0.0)
        return 0
    lax.fori_loop(0, 640 // 16, _dinv, 0)
    pltpu.sync_copy(tbuf, deg_s.at[pl.ds(s * 640, 640)])
    plsc.subcore_barrier()

    pltpu.sync_copy(deg_s, fv)
    def _abatch(k, _):
        off = base_e + k * _EB
        pltpu.sync_copy(row_h.at[pl.ds(off, _EB)], row_b)
        pltpu.sync_copy(av_h.at[pl.ds(c * _E + off, _EB)], w_b)
        def _grp(g, _):
            rv = row_b[pl.ds(g * 16, 16)]
            di = plsc.load_gather(fv, [rv])
            w_b[pl.ds(g * 16, 16)] = w_b[pl.ds(g * 16, 16)] * di
            return 0
        lax.fori_loop(0, _EB // 16, _grp, 0)
        pltpu.sync_copy(w_b, av_h.at[pl.ds(c * _E + off, _EB)])
        return 0
    lax.fori_loop(0, _NB, _abatch, 0)

    def _init_out(j, _):
        r0 = s * _RPT + j * _RC
        g0 = c * _NP + r0
        pltpu.sync_copy(xm2_h.at[pl.ds(g0, _RC)], xm_c)
        pltpu.sync_copy(xm_c, out2_h.at[pl.ds(g0, _RC)])
        pltpu.sync_copy(zf, accum.at[pl.ds(r0, _RC)])
        return 0
    lax.fori_loop(0, _NCH, _init_out, 0)
    plsc.subcore_barrier()

    bufs = ((col_b0, row_b0, a_b0, gath0, sem_g0, sem_i0, sem_s0),
            (col_b1, row_b1, a_b1, gath1, sem_g1, sem_i1, sem_s1))

    def _stage(k, j):
        colp, rowp, ap, gathp, sgp, sip, ssp = bufs[j]
        colq, rowq, aq, gathq, sgq, siq, ssq = bufs[1 - j]
        pltpu.make_async_copy(out2_h.at[colp], gathp, sgp).wait()
        def _scale(g, _):
            av = ap[pl.ds(g * 16, 16)]
            idx_r = g * 16 + _iota16()
            for q in range(_H // 16):
                idx_c = _iota16() + q * 16
                v = plsc.load_gather(gathp, [idx_r, idx_c])
                plsc.store_scatter(gathp, [idx_r, idx_c], v * av)
            return 0
        lax.fori_loop(0, _EB // 16, _scale, 0)
        @pl.when(k > 0)
        def _():
            pltpu.make_async_copy(gathq, accum.at[rowq], ssq).wait()
        @pl.when(k + 1 < _NB)
        def _():
            off1 = base_e + (k + 1) * _EB
            pltpu.async_copy(col_h.at[pl.ds(off1, _EB)], colq, siq)
            pltpu.async_copy(row_h.at[pl.ds(off1, _EB)], rowq, siq)
            pltpu.async_copy(av_h.at[pl.ds(c * _E + off1, _EB)], aq, siq)
            pltpu.make_async_copy(
                col_h.at[pl.ds(off1, _EB)], colq, siq).wait()
            pltpu.make_async_copy(
                row_h.at[pl.ds(off1, _EB)], rowq, siq).wait()
            pltpu.make_async_copy(
                av_h.at[pl.ds(c * _E + off1, _EB)], aq, siq).wait()
            def _adj(g, _):
                colq[pl.ds(g * 16, 16)] = colq[pl.ds(g * 16, 16)] + c * _NP
                return 0
            lax.fori_loop(0, _EB // 16, _adj, 0)
            pltpu.async_copy(out2_h.at[colq], gathq, sgq)
        pltpu.async_copy(gathp, accum.at[rowp], ssp, add=True)

    def _iter(t, _):
        pltpu.sync_copy(col_h.at[pl.ds(base_e, _EB)], col_b0)
        pltpu.sync_copy(row_h.at[pl.ds(base_e, _EB)], row_b0)
        pltpu.sync_copy(av_h.at[pl.ds(c * _E + base_e, _EB)], a_b0)
        def _adj0(g, _):
            col_b0[pl.ds(g * 16, 16)] = col_b0[pl.ds(g * 16, 16)] + c * _NP
            return 0
        lax.fori_loop(0, _EB // 16, _adj0, 0)
        pltpu.async_copy(out2_h.at[col_b0], gath0, sem_g0)

        def _pair(i, _):
            _stage(2 * i, 0)
            _stage(2 * i + 1, 1)
            return 0
        lax.fori_loop(0, _NB // 2, _pair, 0)
        pltpu.make_async_copy(gath1, accum.at[row_b1], sem_s1).wait()
        plsc.subcore_barrier()

        def _cmb(j, _):
            r0 = s * _RPT + j * _RC
            g0 = c * _NP + r0
            pltpu.async_copy(xm2_h.at[pl.ds(g0, _RC)], xm_c, sem)
            pltpu.async_copy(nm2_h.at[pl.ds(g0, _RC)], nm_c, sem)
            pltpu.sync_copy(accum.at[pl.ds(r0, _RC)], acc_c)
            pltpu.make_async_copy(
                xm2_h.at[pl.ds(g0, _RC)], xm_c, sem).wait()
            pltpu.make_async_copy(
                nm2_h.at[pl.ds(g0, _RC)], nm_c, sem).wait()
            pltpu.sync_copy(zf, accum.at[pl.ds(r0, _RC)])
            def _crow(r, _):
                idx_r = jnp.full((16,), r, jnp.int32)
                for q in range(_H // 16):
                    idx_c = _iota16() + q * 16
                    a = plsc.load_gather(acc_c, [idx_r, idx_c])
                    x = plsc.load_gather(xm_c, [idx_r, idx_c])
                    m = plsc.load_gather(nm_c, [idx_r, idx_c])
                    plsc.store_scatter(acc_c, [idx_r, idx_c], x + m * a)
                return 0
            lax.fori_loop(0, _RC, _crow, 0)
            pltpu.sync_copy(acc_c, out2_h.at[pl.ds(g0, _RC)])
            return 0
        lax.fori_loop(0, _NCH, _cmb, 0)
        plsc.subcore_barrier()
        return 0
    lax.fori_loop(0, _NUM_ITERS, _iter, 0)


def _sc_propagate(xm2, nm2, row, col, seed):
    mesh = plsc.VectorSubcoreMesh(core_axis_name="c", subcore_axis_name="s")
    fn = functools.partial(
        pl.kernel, _sc_body, mesh=mesh,
        compiler_params=pltpu.CompilerParams(needs_layout_passes=False,
                                             use_tc_tiling_on_sc=False),
        out_type=[
            jax.ShapeDtypeStruct((2 * _NP, _H), jnp.float32),
            jax.ShapeDtypeStruct((_N,), jnp.float32),
            jax.ShapeDtypeStruct((2 * _E,), jnp.float32),
        ],
        scratch_types=[
            pltpu.VMEM((_NP,), jnp.int32),
            pltpu.VMEM((_NP,), jnp.float32),
            pltpu.VMEM((_NP,), jnp.int32),
            pltpu.VMEM((_EB, _H), jnp.float32),
            pltpu.VMEM((_EB, _H), jnp.float32),
            pltpu.VMEM((_EB,), jnp.int32),
            pltpu.VMEM((_EB,), jnp.int32),
            pltpu.VMEM((_EB,), jnp.int32),
            pltpu.VMEM((_EB,), jnp.int32),
            pltpu.VMEM((_EB,), jnp.float32),
            pltpu.VMEM((_EB,), jnp.float32),
            pltpu.VMEM((_EB,), jnp.int32),
            pltpu.VMEM((_EB,), jnp.float32),
            pltpu.VMEM((640,), jnp.float32),
            pltpu.VMEM((_RC, _H), jnp.float32),
            pltpu.VMEM((_RC, _H), jnp.float32),
            pltpu.VMEM((_RC, _H), jnp.float32),
            pltpu.VMEM((_RC, _H), jnp.float32),
            pltpu.VMEM((640,), jnp.int32),
            pltpu.VMEM((640,), jnp.float32),
            pltpu.VMEM_SHARED((_NP, _H), jnp.float32),
            pltpu.VMEM_SHARED((_NP,), jnp.int32),
            pltpu.VMEM_SHARED((_NP,), jnp.float32),
            pltpu.SemaphoreType.DMA,
            pltpu.SemaphoreType.DMA,
            pltpu.SemaphoreType.DMA,
            pltpu.SemaphoreType.DMA,
            pltpu.SemaphoreType.DMA,
            pltpu.SemaphoreType.DMA,
            pltpu.SemaphoreType.DMA,
        ],
    )()
    out2, f, _ = fn(xm2, nm2, row, col, seed)
    return out2, f


def _corr_body(out_ref, f_ref, o_ref):
    out = out_ref[...]
    f = f_ref[...]
    n, d = out.shape
    mean = jnp.mean(out, axis=0, keepdims=True)
    xm = out - mean
    cov = jnp.dot(xm.T, xm, preferred_element_type=jnp.float32) / (n - 1)
    eye = (lax.broadcasted_iota(jnp.int32, (d, d), 0)
           == lax.broadcasted_iota(jnp.int32, (d, d), 1))
    var = jnp.sum(jnp.where(eye, cov, 0.0), axis=1)
    std = jnp.sqrt(var)
    denom = std[:, None] * std[None, :]
    cor = jnp.where(denom > 0, cov / denom, 0.0)
    cor = jnp.where(eye, 0.0, cor)
    af = jnp.exp(f * _LN_ALPHA)
    a_1 = af * xm
    a_2 = jnp.dot(a_1, cor, preferred_element_type=jnp.float32)
    o_ref[...] = out + _BETA * (1.0 - af) * a_2


def _corr_correction(out, f):
    return pl.pallas_call(
        _corr_body,
        out_shape=jax.ShapeDtypeStruct(out.shape, jnp.float32),
    )(out, f.reshape(-1, 1))


def kernel(x, edge_index, mask):
    row = edge_index[0]
    col = edge_index[1]
    xm = jnp.where(mask, x, 0.0)
    nm = (~mask).astype(jnp.float32)
    pad = jnp.zeros((_NP - _N, _H), jnp.float32)
    xm2 = jnp.concatenate([xm[:, :_H], pad, xm[:, _H:], pad], axis=0)
    nm2 = jnp.concatenate([nm[:, :_H], pad, nm[:, _H:], pad], axis=0)
    seed = mask[:, 0].astype(jnp.int32)
    out2, f = _sc_propagate(xm2, nm2, row, col, seed)
    out = jnp.concatenate([out2[:_N], out2[_NP:_NP + _N]], axis=1)
    return _corr_correction(out, f)
